# baseline (speedup 1.0000x reference)
import jax
import jax.numpy as jnp
from jax.experimental import pallas as pl
from jax.experimental.pallas import tpu as pltpu


def _dma_copy_body(u_src, i_src, u_dst, i_dst, sem_u, sem_i):
    cu = pltpu.make_async_copy(u_src, u_dst, sem_u)
    ci = pltpu.make_async_copy(i_src, i_dst, sem_i)
    cu.start()
    ci.start()
    cu.wait()
    ci.wait()


def kernel(W_user, W_item):
    out = pl.pallas_call(
        _dma_copy_body,
        in_specs=[
            pl.BlockSpec(memory_space=pltpu.HBM),
            pl.BlockSpec(memory_space=pltpu.HBM),
        ],
        out_specs=[
            pl.BlockSpec(memory_space=pltpu.HBM),
            pl.BlockSpec(memory_space=pltpu.HBM),
        ],
        out_shape=[
            jax.ShapeDtypeStruct(W_user.shape, W_user.dtype),
            jax.ShapeDtypeStruct(W_item.shape, W_item.dtype),
        ],
        scratch_shapes=[pltpu.SemaphoreType.DMA, pltpu.SemaphoreType.DMA],
    )(W_user, W_item)
    return (out[0], out[1])


# speedup vs baseline: 12.2335x; 12.2335x over previous
import jax
import jax.numpy as jnp
from jax.experimental import pallas as pl
from jax.experimental.pallas import tpu as pltpu


def _copy_body(src_ref, dst_ref):
    dst_ref[...] = src_ref[...]


def _copy_table(x, block_rows):
    n, d = x.shape
    x2 = x.reshape(n // 2, d * 2)
    m = n // 2
    assert m % block_rows == 0
    out = pl.pallas_call(
        _copy_body,
        grid=(m // block_rows,),
        in_specs=[pl.BlockSpec((block_rows, d * 2), lambda i: (i, 0))],
        out_specs=pl.BlockSpec((block_rows, d * 2), lambda i: (i, 0)),
        out_shape=jax.ShapeDtypeStruct((m, d * 2), x.dtype),
    )(x2)
    return out.reshape(n, d)


def kernel(W_user, W_item):
    return (_copy_table(W_user, 5000), _copy_table(W_item, 5000))


# speedup vs baseline: 16.1254x; 1.3181x over previous
import jax
import jax.numpy as jnp
from jax.experimental import pallas as pl
from jax.experimental.pallas import tpu as pltpu

_R = 10000
_NBUF = 8
_LAG = 2


def _ring_copy_body(u_src, i_src, u_dst, i_dst, buf, sem_in, sem_out):
    chunks = []
    for c in range(100000 // _R):
        chunks.append((u_src, u_dst, c * _R))
    for c in range(1000000 // _R):
        chunks.append((i_src, i_dst, c * _R))
    T = len(chunks)

    def start_in(c):
        s, _, off = chunks[c]
        b = c % _NBUF
        pltpu.make_async_copy(
            s.at[pl.ds(off, _R), :], buf.at[b], sem_in.at[b]
        ).start()

    def wait_in(c):
        s, _, off = chunks[c]
        b = c % _NBUF
        pltpu.make_async_copy(
            s.at[pl.ds(off, _R), :], buf.at[b], sem_in.at[b]
        ).wait()

    def start_out(c):
        _, d, off = chunks[c]
        b = c % _NBUF
        pltpu.make_async_copy(
            buf.at[b], d.at[pl.ds(off, _R), :], sem_out.at[b]
        ).start()

    def wait_out(c):
        _, d, off = chunks[c]
        b = c % _NBUF
        pltpu.make_async_copy(
            buf.at[b], d.at[pl.ds(off, _R), :], sem_out.at[b]
        ).wait()

    out_waited = [False] * T
    for b in range(min(_NBUF, T)):
        start_in(b)
    for c in range(T):
        r = c - _LAG
        if 0 <= r and r + _NBUF < T:
            wait_out(r)
            out_waited[r] = True
            start_in(r + _NBUF)
        wait_in(c)
        start_out(c)
    for c in range(T):
        if not out_waited[c]:
            wait_out(c)


def kernel(W_user, W_item):
    out = pl.pallas_call(
        _ring_copy_body,
        in_specs=[
            pl.BlockSpec(memory_space=pltpu.HBM),
            pl.BlockSpec(memory_space=pltpu.HBM),
        ],
        out_specs=[
            pl.BlockSpec(memory_space=pltpu.HBM),
            pl.BlockSpec(memory_space=pltpu.HBM),
        ],
        out_shape=[
            jax.ShapeDtypeStruct(W_user.shape, W_user.dtype),
            jax.ShapeDtypeStruct(W_item.shape, W_item.dtype),
        ],
        scratch_shapes=[
            pltpu.VMEM((_NBUF, _R, 64), jnp.float32),
            pltpu.SemaphoreType.DMA((_NBUF,)),
            pltpu.SemaphoreType.DMA((_NBUF,)),
        ],
    )(W_user, W_item)
    return (out[0], out[1])
